# baseline (speedup 1.0000x reference)
import functools

import jax
import jax.numpy as jnp
from jax import lax
from jax.experimental import pallas as pl
from jax.experimental.pallas import tpu as pltpu
from jax.experimental.pallas import tpu_sc as plsc

VOCAB = 1000000
BIGRAM_DIM = 64
MODEL_DIM = 1024
BATCH = 4
SEQ = 4096
TOKENS = BATCH * SEQ

NC = 2
NS = 16
NW = NC * NS
CHUNK = TOKENS // NW
GCHUNK = 128
NG = CHUNK // GCHUNK
NVEC = CHUNK // 16

_MOD = VOCAB - 1
_RECIP = 1.0 / _MOD


def _mod999999(x):
    m = jnp.int32(_MOD)
    q = (x.astype(jnp.float32) * jnp.float32(_RECIP)).astype(jnp.int32)
    r = x - q * m
    r = jnp.where(r < 0, r + m, r)
    r = jnp.where(r < 0, r + m, r)
    r = jnp.where(r >= m, r - m, r)
    return r


def _sc_gather(prev_hbm, cur_hbm, table_hbm):
    mesh = plsc.VectorSubcoreMesh(core_axis_name="c", subcore_axis_name="s")

    @functools.partial(
        pl.kernel,
        mesh=mesh,
        compiler_params=pltpu.CompilerParams(use_tc_tiling_on_sc=False),
        out_type=jax.ShapeDtypeStruct((TOKENS, BIGRAM_DIM), jnp.float32),
        scratch_types=[
            pltpu.VMEM((CHUNK,), jnp.int32),
            pltpu.VMEM((CHUNK,), jnp.int32),
            pltpu.VMEM((NG, GCHUNK), jnp.int32),
            pltpu.VMEM((CHUNK, BIGRAM_DIM), jnp.float32),
            pltpu.SemaphoreType.DMA,
        ],
    )
    def body(prev_ref, cur_ref, table_ref, out_ref, prev_v, cur_v, idx_v, rows_v, sem):
        wid = lax.axis_index("s") * NC + lax.axis_index("c")
        base = wid * CHUNK
        pltpu.sync_copy(prev_ref.at[pl.ds(base, CHUNK)], prev_v)
        pltpu.sync_copy(cur_ref.at[pl.ds(base, CHUNK)], cur_v)

        lane = lax.iota(jnp.int32, 16)
        for v in range(NVEC):
            prev = prev_v[pl.ds(v * 16, 16)]
            cur = cur_v[pl.ds(v * 16, 16)]
            h = _mod999999(
                jnp.bitwise_xor(cur * jnp.int32(36313), prev * jnp.int32(27191))
            )
            pos = base + v * 16 + lane
            is_first = (pos & jnp.int32(SEQ - 1)) == 0
            idx = jnp.where(is_first, jnp.int32(_MOD), h)
            idx_v[v // (GCHUNK // 16), pl.ds((v % (GCHUNK // 16)) * 16, 16)] = idx

        copies = []
        for g in range(NG):
            copies.append(
                pltpu.async_copy(
                    table_ref.at[idx_v.at[g]],
                    rows_v.at[pl.ds(g * GCHUNK, GCHUNK)],
                    sem,
                )
            )
        for c in copies:
            c.wait()

        pltpu.sync_copy(rows_v, out_ref.at[pl.ds(base, CHUNK)])

    return body(prev_hbm, cur_hbm, table_hbm)


def _tc_matmul(gathered, proj_weight, scale):
    ROWS = 1024
    grid = TOKENS // ROWS

    def body(scale_ref, g_ref, p_ref, o_ref):
        o_ref[...] = lax.dot_general(
            g_ref[...],
            p_ref[...] * scale_ref[0],
            (((1,), (1,)), ((), ())),
            preferred_element_type=jnp.float32,
        )

    return pl.pallas_call(
        body,
        grid=(grid,),
        in_specs=[
            pl.BlockSpec(memory_space=pltpu.SMEM),
            pl.BlockSpec((ROWS, BIGRAM_DIM), lambda i: (i, 0)),
            pl.BlockSpec((MODEL_DIM, BIGRAM_DIM), lambda i: (0, 0)),
        ],
        out_specs=pl.BlockSpec((ROWS, MODEL_DIM), lambda i: (i, 0)),
        out_shape=jax.ShapeDtypeStruct((TOKENS, MODEL_DIM), jnp.float32),
    )(scale.reshape(1), gathered, proj_weight)


def kernel(token_ids, embed_weight, proj_weight, scale):
    flat = token_ids.astype(jnp.int32).reshape(-1)
    prev = jnp.concatenate([jnp.zeros((1,), jnp.int32), flat[:-1]])
    gathered = _sc_gather(prev, flat, embed_weight)
    out = _tc_matmul(gathered, proj_weight, scale.astype(jnp.float32))
    return out.reshape(BATCH, SEQ, MODEL_DIM)
